# Initial kernel scaffold; baseline (speedup 1.0000x reference)
import jax
import jax.numpy as jnp
from jax.experimental import pallas as pl


def kernel(codes, codes_table, W_in, b_in, W_h1, b_h1, W_h2, b_h2, W_h3, b_h3, W_si2, b_si2, W_si3, b_si3, W_s1o, b_s1o, W_s2o, b_s2o, W_mu, b_mu, W_s, b_s):
    raise NotImplementedError("write your pallas kernel here")


# speedup vs baseline: 3.2320x; 3.2320x over previous
import functools

import jax
import jax.numpy as jnp
from jax import lax
from jax.experimental import pallas as pl
from jax.experimental.pallas import tpu as pltpu
from jax.experimental.pallas import tpu_sc as plsc

B = 1024
N = 100000
D = 64
H = 512

NP = N // 2
MB = 1024
GRID = (NP + MB - 1) // MB


def _argmin_body(codes_t_ref, table_ref, idx_out_ref, best_val, best_idx):
    step = pl.program_id(0)
    blk = table_ref[...]
    pair = step * MB + lax.broadcasted_iota(jnp.int32, (MB, 1), 0)
    valid = pair < NP
    big_i = jnp.int32(2**30)
    ct = codes_t_ref[...]

    def half_scores(h):
        x = blk[:, h * D:(h + 1) * D]
        norm = jnp.sum(x * x, axis=1, keepdims=True)
        dots = jnp.dot(x, ct, preferred_element_type=jnp.float32)
        return jnp.where(valid, norm - 2.0 * dots, jnp.inf)

    s0 = half_scores(0)
    s1 = half_scores(1)
    bmin = jnp.minimum(jnp.min(s0, axis=0, keepdims=True),
                       jnp.min(s1, axis=0, keepdims=True))
    c0 = jnp.min(jnp.where(s0 == bmin, 2 * pair, big_i), axis=0, keepdims=True)
    c1 = jnp.min(jnp.where(s1 == bmin, 2 * pair + 1, big_i), axis=0, keepdims=True)
    bidx = jnp.minimum(c0, c1)

    @pl.when(step == 0)
    def _():
        best_val[...] = bmin
        best_idx[...] = bidx

    @pl.when(step > 0)
    def _():
        prev_val = best_val[...]
        prev_idx = best_idx[...]
        pred = bmin < prev_val
        best_val[...] = jnp.where(pred, bmin, prev_val)
        best_idx[...] = jnp.where(pred, bidx, prev_idx)

    @pl.when(step == GRID - 1)
    def _():
        idx_out_ref[...] = best_idx[...]


def _nearest_indices(codes, table2):
    codes_t = codes.T
    idx = pl.pallas_call(
        _argmin_body,
        grid=(GRID,),
        in_specs=[
            pl.BlockSpec((D, B), lambda i: (0, 0)),
            pl.BlockSpec((MB, 2 * D), lambda i: (i, 0)),
        ],
        out_specs=pl.BlockSpec((1, B), lambda i: (0, 0)),
        out_shape=jax.ShapeDtypeStruct((1, B), jnp.int32),
        scratch_shapes=[
            pltpu.VMEM((1, B), jnp.float32),
            pltpu.VMEM((1, B), jnp.int32),
        ],
    )(codes_t, table2)
    return idx.reshape(B)


def _gather_rows(table2, idx):
    info = plsc.get_sparse_core_info()
    nw = info.num_cores * info.num_subcores
    b_per_w = B // nw
    lanes = info.num_lanes
    mesh = plsc.VectorSubcoreMesh(core_axis_name="c", subcore_axis_name="s")

    @functools.partial(
        pl.kernel,
        mesh=mesh,
        out_type=jax.ShapeDtypeStruct((B, 2 * D), jnp.float32),
        scratch_types=[
            pltpu.VMEM((b_per_w,), jnp.int32),
            pltpu.VMEM((b_per_w,), jnp.int32),
            pltpu.VMEM((b_per_w, 2 * D), jnp.float32),
            pltpu.SemaphoreType.DMA,
        ],
    )
    def gather(table_hbm, idx_hbm, out_hbm, idx_v, idx2_v, rows_v, sem):
        wid = lax.axis_index("s") * info.num_cores + lax.axis_index("c")
        base = wid * b_per_w
        pltpu.sync_copy(idx_hbm.at[pl.ds(base, b_per_w)], idx_v)
        for j in range(b_per_w // lanes):
            sl = pl.ds(j * lanes, lanes)
            idx2_v[sl] = lax.shift_right_logical(idx_v[sl], 1)
        pltpu.async_copy(table_hbm.at[idx2_v], rows_v, sem).wait()
        pltpu.sync_copy(rows_v, out_hbm.at[pl.ds(base, b_per_w)])

    return gather(table2, idx)


def _mlp_body(pair_ref, idx_ref,
              W_in, b_in, W_h1, b_h1, W_h2, b_h2, W_h3, b_h3,
              W_si2, b_si2, W_si3, b_si3, W_s1o, b_s1o, W_s2o, b_s2o,
              W_mu, b_mu, W_s, b_s,
              mu_ref, logstd_ref):
    def fc(a, w, b):
        return jnp.tanh(
            jnp.dot(a, w[...], preferred_element_type=jnp.float32) + b[...])

    pairs = pair_ref[...]
    parity = idx_ref[...] & 1
    x = jnp.where(parity == 0, pairs[:, :D], pairs[:, D:])
    i = fc(x, W_in, b_in)
    _h1 = fc(i, W_h1, b_h1)
    _s2 = fc(_h1, W_si2, b_si2)
    _s3 = fc(_h1, W_si3, b_si3)
    _h2 = fc(_h1 + _s2, W_h2, b_h2)
    _o1 = fc(_h1, W_s1o, b_s1o)
    _o2 = fc(_h2, W_s2o, b_s2o)
    _o3 = fc(_h2 + _s3, W_h3, b_h3)
    out = _o1 + _o2 + _o3
    mu_ref[...] = (jnp.dot(out, W_mu[...], preferred_element_type=jnp.float32)
                   + b_mu[...])
    logstd_ref[...] = (jnp.dot(out, W_s[...], preferred_element_type=jnp.float32)
                       + b_s[...])


def _mlp(pairs, idx_col, *weights):
    return pl.pallas_call(
        _mlp_body,
        out_shape=(
            jax.ShapeDtypeStruct((B, D), jnp.float32),
            jax.ShapeDtypeStruct((B, D), jnp.float32),
        ),
    )(pairs, idx_col, *weights)


def kernel(codes, codes_table, W_in, b_in, W_h1, b_h1, W_h2, b_h2, W_h3, b_h3,
           W_si2, b_si2, W_si3, b_si3, W_s1o, b_s1o, W_s2o, b_s2o,
           W_mu, b_mu, W_s, b_s):
    table2 = codes_table.reshape(NP, 2 * D)
    idx = _nearest_indices(codes, table2)
    pairs = _gather_rows(table2, idx)
    mu, logstd = _mlp(pairs, idx.reshape(B, 1),
                      W_in, b_in, W_h1, b_h1, W_h2, b_h2, W_h3, b_h3,
                      W_si2, b_si2, W_si3, b_si3, W_s1o, b_s1o, W_s2o, b_s2o,
                      W_mu, b_mu, W_s, b_s)
    return (mu, logstd)


# speedup vs baseline: 3.8500x; 1.1912x over previous
import functools

import jax
import jax.numpy as jnp
from jax import lax
from jax.experimental import pallas as pl
from jax.experimental.pallas import tpu as pltpu
from jax.experimental.pallas import tpu_sc as plsc

B = 1024
N = 100000
D = 64
H = 512

NP = N // 2
MB = 1000
GRID = NP // MB


def _argmin_body(codes_t2_ref, table_ref, idx_out_ref, best_val, best_idx):
    step = pl.program_id(0)
    blk = table_ref[...]
    ct2 = codes_t2_ref[...]

    def half_scores(h):
        x = blk[:, h * D:(h + 1) * D]
        norm = jnp.sum(x * x, axis=1, keepdims=True)
        dots = jnp.dot(x, ct2, preferred_element_type=jnp.float32)
        return norm + dots

    s = jnp.minimum(half_scores(0), half_scores(1))
    bmin = jnp.min(s, axis=0, keepdims=True)
    pair = step * MB + lax.broadcasted_iota(jnp.int32, (MB, 1), 0)
    bidx = jnp.min(jnp.where(s == bmin, pair, jnp.int32(2**30)),
                   axis=0, keepdims=True)

    @pl.when(step == 0)
    def _():
        best_val[...] = bmin
        best_idx[...] = bidx

    @pl.when(step > 0)
    def _():
        prev_val = best_val[...]
        prev_idx = best_idx[...]
        pred = bmin < prev_val
        best_val[...] = jnp.where(pred, bmin, prev_val)
        best_idx[...] = jnp.where(pred, bidx, prev_idx)

    @pl.when(step == GRID - 1)
    def _():
        idx_out_ref[...] = best_idx[...]


def _nearest_pairs(codes_t2, table2):
    idx = pl.pallas_call(
        _argmin_body,
        grid=(GRID,),
        in_specs=[
            pl.BlockSpec((D, B), lambda i: (0, 0)),
            pl.BlockSpec((MB, 2 * D), lambda i: (i, 0)),
        ],
        out_specs=pl.BlockSpec((1, B), lambda i: (0, 0)),
        out_shape=jax.ShapeDtypeStruct((1, B), jnp.int32),
        scratch_shapes=[
            pltpu.VMEM((1, B), jnp.float32),
            pltpu.VMEM((1, B), jnp.int32),
        ],
    )(codes_t2, table2)
    return idx.reshape(B)


def _gather_rows(table2, idx):
    info = plsc.get_sparse_core_info()
    nw = info.num_cores * info.num_subcores
    b_per_w = B // nw
    mesh = plsc.VectorSubcoreMesh(core_axis_name="c", subcore_axis_name="s")

    @functools.partial(
        pl.kernel,
        mesh=mesh,
        out_type=jax.ShapeDtypeStruct((B, 2 * D), jnp.float32),
        scratch_types=[
            pltpu.VMEM((b_per_w,), jnp.int32),
            pltpu.VMEM((b_per_w, 2 * D), jnp.float32),
            pltpu.SemaphoreType.DMA,
        ],
    )
    def gather(table_hbm, idx_hbm, out_hbm, idx_v, rows_v, sem):
        wid = lax.axis_index("s") * info.num_cores + lax.axis_index("c")
        base = wid * b_per_w
        pltpu.sync_copy(idx_hbm.at[pl.ds(base, b_per_w)], idx_v)
        pltpu.async_copy(table_hbm.at[idx_v], rows_v, sem).wait()
        pltpu.sync_copy(rows_v, out_hbm.at[pl.ds(base, b_per_w)])

    return gather(table2, idx)


def _mlp_body(pair_ref, codes_ref,
              W_in, b_in, W_h1, b_h1, W_h2, b_h2, W_h3, b_h3,
              W_si2, b_si2, W_si3, b_si3, W_s1o, b_s1o, W_s2o, b_s2o,
              W_mu, b_mu, W_s, b_s,
              mu_ref, logstd_ref):
    def fc(a, w, b):
        return jnp.tanh(
            jnp.dot(a, w[...], preferred_element_type=jnp.float32) + b[...])

    pairs = pair_ref[...]
    q = codes_ref[...]
    e, o = pairs[:, :D], pairs[:, D:]
    d0 = jnp.sum((e - q) * (e - q), axis=1, keepdims=True)
    d1 = jnp.sum((o - q) * (o - q), axis=1, keepdims=True)
    x = jnp.where(d0 <= d1, e, o)
    i = fc(x, W_in, b_in)
    _h1 = fc(i, W_h1, b_h1)
    _s2 = fc(_h1, W_si2, b_si2)
    _s3 = fc(_h1, W_si3, b_si3)
    _h2 = fc(_h1 + _s2, W_h2, b_h2)
    _o1 = fc(_h1, W_s1o, b_s1o)
    _o2 = fc(_h2, W_s2o, b_s2o)
    _o3 = fc(_h2 + _s3, W_h3, b_h3)
    out = _o1 + _o2 + _o3
    mu_ref[...] = (jnp.dot(out, W_mu[...], preferred_element_type=jnp.float32)
                   + b_mu[...])
    logstd_ref[...] = (jnp.dot(out, W_s[...], preferred_element_type=jnp.float32)
                       + b_s[...])


def _mlp(pairs, codes, *weights):
    return pl.pallas_call(
        _mlp_body,
        out_shape=(
            jax.ShapeDtypeStruct((B, D), jnp.float32),
            jax.ShapeDtypeStruct((B, D), jnp.float32),
        ),
    )(pairs, codes, *weights)


def kernel(codes, codes_table, W_in, b_in, W_h1, b_h1, W_h2, b_h2, W_h3, b_h3,
           W_si2, b_si2, W_si3, b_si3, W_s1o, b_s1o, W_s2o, b_s2o,
           W_mu, b_mu, W_s, b_s):
    table2 = codes_table.reshape(NP, 2 * D)
    codes_t2 = -2.0 * codes.T
    idx = _nearest_pairs(codes_t2, table2)
    pairs = _gather_rows(table2, idx)
    mu, logstd = _mlp(pairs, codes,
                      W_in, b_in, W_h1, b_h1, W_h2, b_h2, W_h3, b_h3,
                      W_si2, b_si2, W_si3, b_si3, W_s1o, b_s1o, W_s2o, b_s2o,
                      W_mu, b_mu, W_s, b_s)
    return (mu, logstd)
